# baseline (speedup 1.0000x reference)
import functools

import jax
import jax.numpy as jnp
from jax import lax
from jax.experimental import pallas as pl
from jax.experimental.pallas import tpu as pltpu
from jax.experimental.pallas import tpu_sc as plsc

B, C, D = 64, 768, 576
RN = B * D
NC, NS = 2, 16
NW = NC * NS
CH = 32
RPW = RN // NW
NCH = RPW // CH
NJ = C // 16


@functools.partial(
    pl.kernel,
    mesh=plsc.VectorSubcoreMesh(core_axis_name="c", subcore_axis_name="s"),
    out_type=jax.ShapeDtypeStruct((RN, C), jnp.float32),
    scratch_types=[
        pltpu.VMEM((NCH, CH), jnp.int32),
        pltpu.VMEM((NJ, 16), jnp.int32),
        pltpu.VMEM((CH, C), jnp.float32),
        pltpu.VMEM((CH, C), jnp.float32),
        pltpu.VMEM((CH, C), jnp.float32),
        pltpu.VMEM((CH, C), jnp.float32),
        pltpu.SemaphoreType.DMA,
        pltpu.SemaphoreType.DMA,
        pltpu.SemaphoreType.DMA,
        pltpu.SemaphoreType.DMA,
    ],
    compiler_params=pltpu.CompilerParams(needs_layout_passes=False),
)
def _sc_permute_lanes(x_hbm, ridx_hbm, pcol_hbm, out_hbm,
                      ridx_v, pcol_v, vb0, vb1, ob0, ob1,
                      gs0, gs1, ss0, ss1):
    wid = lax.axis_index("s") * NC + lax.axis_index("c")
    pltpu.sync_copy(ridx_hbm.at[wid], ridx_v)
    pltpu.sync_copy(pcol_hbm, pcol_v)

    vbufs = (vb0, vb1)
    obufs = (ob0, ob1)
    gsems = (gs0, gs1)
    ssems = (ss0, ss1)

    def start_gather(chunk, b):
        pltpu.async_copy(x_hbm.at[ridx_v.at[chunk]], vbufs[b], gsems[b])

    def drain(buf, sem):
        pltpu.make_async_copy(x_hbm.at[pl.ds(0, CH)], buf, sem).wait()

    def permute(b):
        vb, ob = vbufs[b], obufs[b]

        def row_body(r, carry):
            for j in range(NJ):
                pc = pcol_v[j]
                v = plsc.load_gather(vb, [jnp.full((16,), r, jnp.int32), pc])
                ob[r, pl.ds(j * 16, 16)] = v
            return carry

        lax.fori_loop(0, CH, row_body, 0, unroll=False)

    def start_scatter(chunk, b):
        pltpu.async_copy(obufs[b], out_hbm.at[ridx_v.at[chunk]], ssems[b])

    def process(chunk, b, wait_scatter):
        drain(vbufs[b], gsems[b])
        if wait_scatter:
            drain(obufs[b], ssems[b])
        permute(b)
        start_scatter(chunk, b)

    start_gather(0, 0)
    start_gather(1, 1)
    process(0, 0, False)
    start_gather(2, 0)
    process(1, 1, False)
    start_gather(3, 1)

    def pair_body(p, carry):
        for b in range(2):
            cur = 2 * p + b
            drain(vbufs[b], gsems[b])
            drain(obufs[b], ssems[b])
            permute(b)
            start_scatter_cur(cur, b)
            nxt = cur + 2

            @pl.when(nxt < NCH)
            def _gather_next(b=b, nxt=nxt):
                pltpu.async_copy(x_hbm.at[ridx_v.at[nxt]], vbufs[b],
                                 gsems[b])
        return carry

    def start_scatter_cur(cur, b):
        pltpu.async_copy(obufs[b], out_hbm.at[ridx_v.at[cur]], ssems[b])

    lax.fori_loop(1, NCH // 2, pair_body, 0, unroll=False)

    drain(obufs[0], ssems[0])
    drain(obufs[1], ssems[1])


def kernel(inp):
    perm = jax.random.permutation(jax.random.key(1), C).astype(jnp.int32)
    pcol = perm.reshape(NJ, 16)
    ridx = jnp.arange(RN, dtype=jnp.int32).reshape(NW, NCH, CH)
    x2d = jnp.transpose(inp, (0, 2, 1)).reshape(RN, C)
    out2d = _sc_permute_lanes(x2d, ridx, pcol)
    return out2d.reshape(B, D, C).transpose(0, 2, 1)


# speedup vs baseline: 1.8131x; 1.8131x over previous
import functools

import jax
import jax.numpy as jnp
from jax import lax
from jax.experimental import pallas as pl
from jax.experimental.pallas import tpu as pltpu
from jax.experimental.pallas import tpu_sc as plsc

B, C, D = 64, 768, 576
RN = B * D
NC, NS = 2, 16
NW = NC * NS
CH = 16
RPW = RN // NW
NCH = RPW // CH
NJ = C // 16


@functools.partial(
    pl.kernel,
    mesh=plsc.VectorSubcoreMesh(core_axis_name="c", subcore_axis_name="s"),
    out_type=jax.ShapeDtypeStruct((RN, C), jnp.float32),
    scratch_types=[
        pltpu.VMEM((NCH, CH), jnp.int32),
        pltpu.VMEM((NJ, 16), jnp.int32),
        pltpu.VMEM((CH, 1024), jnp.float32),
        pltpu.VMEM((CH, 1024), jnp.float32),
        pltpu.VMEM((CH, C), jnp.float32),
        pltpu.VMEM((CH, C), jnp.float32),
        pltpu.SemaphoreType.DMA,
        pltpu.SemaphoreType.DMA,
        pltpu.SemaphoreType.DMA,
        pltpu.SemaphoreType.DMA,
    ],
    compiler_params=pltpu.CompilerParams(needs_layout_passes=False),
)
def _sc_permute_lanes(x_hbm, ridx_hbm, pcol_hbm, out_hbm,
                      ridx_v, pcol_v, vb0, vb1, ob0, ob1,
                      gs0, gs1, ss0, ss1):
    wid = lax.axis_index("s") * NC + lax.axis_index("c")
    pltpu.sync_copy(ridx_hbm.at[wid], ridx_v)
    pltpu.sync_copy(pcol_hbm, pcol_v)

    vbufs = (vb0, vb1)
    obufs = (ob0, ob1)
    gsems = (gs0, gs1)
    ssems = (ss0, ss1)

    def start_gather(chunk, b):
        pltpu.async_copy(x_hbm.at[ridx_v.at[chunk]],
                         vbufs[b].at[:, pl.ds(0, C)], gsems[b])

    def start_scatter(chunk, b):
        pltpu.async_copy(obufs[b], out_hbm.at[ridx_v.at[chunk]], ssems[b])

    def drain_g(b):
        pltpu.make_async_copy(x_hbm.at[pl.ds(0, CH)],
                              vbufs[b].at[:, pl.ds(0, C)], gsems[b]).wait()

    def drain_s(b):
        pltpu.make_async_copy(x_hbm.at[pl.ds(0, CH)], obufs[b],
                              ssems[b]).wait()

    def permute(b):
        vb, ob = vbufs[b], obufs[b]
        for half in range(3):
            pcs = [pcol_v[half * 16 + t] for t in range(16)]
            for k in range(CH):
                i0 = jnp.full((16,), k, jnp.int32)
                for t in range(16):
                    j = half * 16 + t
                    ob[k, pl.ds(j * 16, 16)] = plsc.load_gather(
                        vb, [i0, pcs[t]])

    start_gather(0, 0)
    start_gather(1, 1)

    def step(p, carry):
        for b in range(2):
            cur = 2 * p + b
            drain_g(b)

            @pl.when(cur >= 2)
            def _wait_prev_scatter(b=b):
                drain_s(b)

            permute(b)
            start_scatter(cur, b)

            @pl.when(cur + 2 < NCH)
            def _gather_next(b=b, cur=cur):
                start_gather(cur + 2, b)
        return carry

    lax.fori_loop(0, NCH // 2, step, 0, unroll=False)
    drain_s(0)
    drain_s(1)


def kernel(inp):
    perm = jax.random.permutation(jax.random.key(1), C).astype(jnp.int32)
    pcol = perm.reshape(NJ, 16)
    ridx = jnp.arange(RN, dtype=jnp.int32).reshape(NW, NCH, CH)
    x2d = jnp.transpose(inp, (0, 2, 1)).reshape(RN, C)
    out2d = _sc_permute_lanes(x2d, ridx, pcol)
    return out2d.reshape(B, D, C).transpose(0, 2, 1)


# speedup vs baseline: 1.9787x; 1.0913x over previous
import functools

import jax
import jax.numpy as jnp
from jax import lax
from jax.experimental import pallas as pl
from jax.experimental.pallas import tpu as pltpu
from jax.experimental.pallas import tpu_sc as plsc

B, C, D = 64, 768, 576
RN = B * D
NC, NS = 2, 16
NW = NC * NS
CH = 8
RPW = RN // NW
NCH = RPW // CH
NJ = C // 16


@functools.partial(
    pl.kernel,
    mesh=plsc.VectorSubcoreMesh(core_axis_name="c", subcore_axis_name="s"),
    out_type=jax.ShapeDtypeStruct((RN, C), jnp.float32),
    scratch_types=[
        pltpu.VMEM((NCH, CH), jnp.int32),
        pltpu.VMEM((NJ, 16), jnp.int32),
        pltpu.VMEM((CH, 1024), jnp.float32),
        pltpu.VMEM((CH, 1024), jnp.float32),
        pltpu.VMEM((CH, C), jnp.float32),
        pltpu.VMEM((CH, C), jnp.float32),
        pltpu.SemaphoreType.DMA,
        pltpu.SemaphoreType.DMA,
        pltpu.SemaphoreType.DMA,
        pltpu.SemaphoreType.DMA,
    ],
    compiler_params=pltpu.CompilerParams(needs_layout_passes=False),
)
def _sc_permute_lanes(x_hbm, ridx_hbm, pcol_hbm, out_hbm,
                      ridx_v, pcol_v, vb0, vb1, ob0, ob1,
                      gs0, gs1, ss0, ss1):
    wid = lax.axis_index("s") * NC + lax.axis_index("c")
    pltpu.sync_copy(ridx_hbm.at[wid], ridx_v)
    pltpu.sync_copy(pcol_hbm, pcol_v)

    vbufs = (vb0, vb1)
    obufs = (ob0, ob1)
    gsems = (gs0, gs1)
    ssems = (ss0, ss1)

    def start_gather(chunk, b):
        pltpu.async_copy(x_hbm.at[ridx_v.at[chunk]],
                         vbufs[b].at[:, pl.ds(0, C)], gsems[b])

    def start_scatter(chunk, b):
        pltpu.async_copy(obufs[b], out_hbm.at[ridx_v.at[chunk]], ssems[b])

    def drain_g(b):
        pltpu.make_async_copy(x_hbm.at[pl.ds(0, CH)],
                              vbufs[b].at[:, pl.ds(0, C)], gsems[b]).wait()

    def drain_s(b):
        pltpu.make_async_copy(x_hbm.at[pl.ds(0, CH)], obufs[b],
                              ssems[b]).wait()

    def permute(b):
        vb, ob = vbufs[b], obufs[b]
        for half in range(3):
            pcs = [pcol_v[half * 16 + t] for t in range(16)]
            for k in range(CH):
                i0 = jnp.full((16,), k, jnp.int32)
                for t in range(16):
                    j = half * 16 + t
                    ob[k, pl.ds(j * 16, 16)] = plsc.load_gather(
                        vb, [i0, pcs[t]])

    start_gather(0, 0)
    start_gather(1, 1)

    def step(p, carry):
        for b in range(2):
            cur = 2 * p + b
            drain_g(b)

            @pl.when(cur >= 2)
            def _wait_prev_scatter(b=b):
                drain_s(b)

            permute(b)
            start_scatter(cur, b)

            @pl.when(cur + 2 < NCH)
            def _gather_next(b=b, cur=cur):
                start_gather(cur + 2, b)
        return carry

    lax.fori_loop(0, NCH // 2, step, 0, unroll=False)
    drain_s(0)
    drain_s(1)


def kernel(inp):
    perm = jax.random.permutation(jax.random.key(1), C).astype(jnp.int32)
    pcol = perm.reshape(NJ, 16)
    ridx = jnp.arange(RN, dtype=jnp.int32).reshape(NW, NCH, CH)
    x2d = jnp.transpose(inp, (0, 2, 1)).reshape(RN, C)
    out2d = _sc_permute_lanes(x2d, ridx, pcol)
    return out2d.reshape(B, D, C).transpose(0, 2, 1)


# speedup vs baseline: 6.5491x; 3.3098x over previous
import functools

import jax
import jax.numpy as jnp
from jax import lax
from jax.experimental import pallas as pl
from jax.experimental.pallas import tpu as pltpu
from jax.experimental.pallas import tpu_sc as plsc

B, C, D = 64, 768, 576
RN = B * D
NC, NS = 2, 16
NW = NC * NS
CH = 32
RPW = RN // NW
NCH = RPW // CH
NJ = C // 16


@functools.partial(
    pl.kernel,
    mesh=plsc.VectorSubcoreMesh(core_axis_name="c", subcore_axis_name="s"),
    out_type=jax.ShapeDtypeStruct((RN, C), jnp.float32),
    scratch_types=[
        pltpu.VMEM((NCH, CH), jnp.int32),
        pltpu.VMEM((NJ, 16), jnp.int32),
        pltpu.VMEM((CH, 1024), jnp.float32),
        pltpu.VMEM((CH, 1024), jnp.float32),
        pltpu.VMEM((CH, C), jnp.float32),
        pltpu.VMEM((CH, C), jnp.float32),
        pltpu.SemaphoreType.DMA,
        pltpu.SemaphoreType.DMA,
        pltpu.SemaphoreType.DMA,
        pltpu.SemaphoreType.DMA,
    ],
    compiler_params=pltpu.CompilerParams(needs_layout_passes=False),
)
def _sc_permute_lanes(x_hbm, ridx_hbm, pcol_hbm, out_hbm,
                      ridx_v, pcol_v, vb0, vb1, ob0, ob1,
                      gs0, gs1, ss0, ss1):
    wid = lax.axis_index("s") * NC + lax.axis_index("c")
    pltpu.sync_copy(ridx_hbm.at[wid], ridx_v)
    pltpu.sync_copy(pcol_hbm, pcol_v)

    vbufs = (vb0, vb1)
    obufs = (ob0, ob1)
    gsems = (gs0, gs1)
    ssems = (ss0, ss1)

    def start_gather(chunk, b):
        pltpu.async_copy(x_hbm.at[ridx_v.at[chunk]],
                         vbufs[b].at[:, pl.ds(0, C)], gsems[b])

    def start_scatter(chunk, b):
        pltpu.async_copy(obufs[b], out_hbm.at[ridx_v.at[chunk]], ssems[b])

    def drain_g(b):
        pltpu.make_async_copy(x_hbm.at[pl.ds(0, CH)],
                              vbufs[b].at[:, pl.ds(0, C)], gsems[b]).wait()

    def drain_s(b):
        pltpu.make_async_copy(x_hbm.at[pl.ds(0, CH)], obufs[b],
                              ssems[b]).wait()

    def permute(b):
        vb, ob = vbufs[b], obufs[b]
        for half in range(3):
            pcs = [pcol_v[half * 16 + t] for t in range(16)]

            def row_body(r, pcs=pcs, half=half, vb=vb, ob=ob):
                i0 = jnp.full((16,), r, jnp.int32)
                for t in range(16):
                    j = half * 16 + t
                    ob[r, pl.ds(j * 16, 16)] = plsc.load_gather(
                        vb, [i0, pcs[t]])

            plsc.parallel_loop(0, CH, 1)(row_body)

    start_gather(0, 0)
    start_gather(1, 1)

    def step(p, carry):
        for b in range(2):
            cur = 2 * p + b
            drain_g(b)

            @pl.when(cur >= 2)
            def _wait_prev_scatter(b=b):
                drain_s(b)

            permute(b)
            start_scatter(cur, b)

            @pl.when(cur + 2 < NCH)
            def _gather_next(b=b, cur=cur):
                start_gather(cur + 2, b)
        return carry

    lax.fori_loop(0, NCH // 2, step, 0, unroll=False)
    drain_s(0)
    drain_s(1)


def kernel(inp):
    perm = jax.random.permutation(jax.random.key(1), C).astype(jnp.int32)
    pcol = perm.reshape(NJ, 16)
    ridx = jnp.arange(RN, dtype=jnp.int32).reshape(NW, NCH, CH)
    x2d = jnp.transpose(inp, (0, 2, 1)).reshape(RN, C)
    out2d = _sc_permute_lanes(x2d, ridx, pcol)
    return out2d.reshape(B, D, C).transpose(0, 2, 1)
